# baseline (speedup 1.0000x reference)
import functools

import jax
import jax.numpy as jnp
from jax.experimental import pallas as pl
from jax.experimental.pallas import tpu as pltpu

N = 10000
G = 64
D_IN = 256
HIDDEN = 256
NUM_CLASSES = 10
EPS = 1e-5

BR = 2000
NB = N // BR

_DN_ROWS = (((0,), (0,)), ((), ()))


def _fused_kernel(x_hbm, batch_ref, w1_ref, g1_ref, b1_ref, w2_ref, g2_ref,
                  b2_ref, wfc_ref, bfc_ref, wfc1t_ref, bfc1_ref, out_ref,
                  xv_ref, sem):
    fN = jnp.float32(N)
    bf16 = jnp.bfloat16

    copies = [
        pltpu.make_async_copy(
            x_hbm.at[pl.ds(i * BR, BR), :],
            xv_ref.at[pl.ds(i * BR, BR), :],
            sem.at[i],
        )
        for i in range(NB)
    ]
    for c in copies:
        c.start()

    b2d = batch_ref[...].reshape(1, N)
    ohT = (b2d == jax.lax.broadcasted_iota(jnp.int32, (G, N), 0)).astype(bf16)
    cnt = jnp.sum(ohT.astype(jnp.float32), axis=1, keepdims=True)

    w1b = w1_ref[...].astype(bf16)
    onesb = jnp.ones((BR, 8), bf16)
    hb_blocks = []
    q1 = None
    csx = None
    for i in range(NB):
        copies[i].wait()
        xf = xv_ref[pl.ds(i * BR, BR), :]
        hb = jnp.dot(xf.astype(bf16), w1b,
                     preferred_element_type=jnp.float32).astype(bf16)
        hb_blocks.append(hb)
        q = jax.lax.dot_general(onesb, hb * hb, _DN_ROWS,
                                preferred_element_type=jnp.float32)[0:1, :]
        cs = jnp.sum(xf, axis=0, keepdims=True)
        q1 = q if q1 is None else q1 + q
        csx = cs if csx is None else csx + cs

    mu1 = jnp.dot(csx / fN, w1_ref[...], preferred_element_type=jnp.float32)
    var1 = q1 / fN - mu1 * mu1
    a1f = g1_ref[...] * jax.lax.rsqrt(var1 + EPS)
    a1 = a1f.astype(bf16)
    c1 = (b1_ref[...] - mu1 * a1f).astype(bf16)
    hn1 = jnp.maximum(jnp.concatenate(hb_blocks, axis=0) * a1 + c1,
                      bf16(0.0))

    h2 = jnp.dot(hn1, w2_ref[...].astype(bf16),
                 preferred_element_type=jnp.float32)
    hb2 = h2.astype(bf16)
    cs1 = jnp.sum(hn1.astype(jnp.float32), axis=0, keepdims=True)
    mu2 = jnp.dot(cs1 / fN, w2_ref[...], preferred_element_type=jnp.float32)
    ones8 = jnp.ones((N, 8), bf16)
    q2 = jax.lax.dot_general(ones8, hb2 * hb2, _DN_ROWS,
                             preferred_element_type=jnp.float32)[0:1, :]
    var2 = q2 / fN - mu2 * mu2
    a2f = g2_ref[...] * jax.lax.rsqrt(var2 + EPS)
    a2 = a2f.astype(bf16)
    c2 = (b2_ref[...] - mu2 * a2f).astype(bf16)
    hn2 = jnp.maximum(hb2 * a2 + c2, bf16(0.0))

    dn_mm = (((1,), (0,)), ((), ()))
    sums = jax.lax.dot_general(ohT, hn2, dn_mm,
                               preferred_element_type=jnp.float32)
    pooled = sums / jnp.maximum(cnt, 1.0)

    h3 = jnp.maximum(
        jnp.dot(pooled, wfc_ref[...], preferred_element_type=jnp.float32)
        + bfc_ref[...], 0.0)
    dn_t = (((1,), (1,)), ((), ()))
    logits = jax.lax.dot_general(h3, wfc1t_ref[...], dn_t,
                                 preferred_element_type=jnp.float32)
    logits = logits + bfc1_ref[...]
    m = jnp.max(logits, axis=-1, keepdims=True)
    sh = logits - m
    lse = jnp.log(jnp.sum(jnp.exp(sh), axis=-1, keepdims=True))
    out_ref[...] = (sh - lse).T


@functools.partial(jax.jit, static_argnames=("interpret",))
def _run(x, batch, Wg1, g1, b1, Wg2, g2, b2, Wfc, bfc, Wfc1, bfc1,
         interpret=False):
    in_specs = [pl.BlockSpec(memory_space=pl.ANY)] + [
        pl.BlockSpec(memory_space=pltpu.MemorySpace.VMEM) for _ in range(11)
    ]
    out_t = pl.pallas_call(
        _fused_kernel,
        in_specs=in_specs,
        out_specs=pl.BlockSpec(memory_space=pltpu.MemorySpace.VMEM),
        out_shape=jax.ShapeDtypeStruct((NUM_CLASSES, G), jnp.float32),
        scratch_shapes=[
            pltpu.VMEM((N, D_IN), jnp.float32),
            pltpu.SemaphoreType.DMA((NB,)),
        ],
        interpret=interpret,
    )(x, batch, Wg1, g1.reshape(1, HIDDEN), b1.reshape(1, HIDDEN),
      Wg2, g2.reshape(1, HIDDEN), b2.reshape(1, HIDDEN),
      Wfc, bfc.reshape(1, HIDDEN), Wfc1.T, bfc1.reshape(1, NUM_CLASSES))
    return out_t.T


def kernel(x, edge_index, batch, Wg1, bg1, g1, b1, Wg2, bg2, g2, b2,
           Wfc, bfc, Wfc1, bfc1):
    del edge_index, bg1, bg2
    return _run(x, batch, Wg1, g1, b1, Wg2, g2, b2, Wfc, bfc, Wfc1, bfc1)


# speedup vs baseline: 1.0225x; 1.0225x over previous
import functools

import jax
import jax.numpy as jnp
from jax.experimental import pallas as pl

N = 10000
G = 64
D_IN = 256
HIDDEN = 256
NUM_CLASSES = 10
EPS = 1e-5

_DN_ROWS = (((0,), (0,)), ((), ()))


def _colsum_sq(v, ones8):
    return jax.lax.dot_general(ones8, v * v, _DN_ROWS,
                               preferred_element_type=jnp.float32)[0:1, :]


def _fused_kernel(x_ref, batch_ref, w1_ref, g1_ref, b1_ref, w2_ref, g2_ref,
                  b2_ref, wfc_ref, bfc_ref, wfc1t_ref, bfc1_ref, out_ref):
    fN = jnp.float32(N)
    bf16 = jnp.bfloat16
    ones8 = jnp.ones((N, 8), bf16)

    xb = x_ref[...].astype(bf16)
    w1 = w1_ref[...].astype(bf16)
    h1 = jnp.dot(xb, w1, preferred_element_type=jnp.float32)
    hb1 = h1.astype(bf16)
    csx = jnp.sum(x_ref[...], axis=0, keepdims=True)
    mu1 = jnp.dot(csx / fN, w1_ref[...], preferred_element_type=jnp.float32)
    var1 = _colsum_sq(hb1, ones8) / fN - mu1 * mu1
    a1f = g1_ref[...] * jax.lax.rsqrt(var1 + EPS)
    a1 = a1f.astype(bf16)
    c1 = (b1_ref[...] - mu1 * a1f).astype(bf16)
    hn1 = jnp.maximum(hb1 * a1 + c1, bf16(0.0))

    h2 = jnp.dot(hn1, w2_ref[...].astype(bf16),
                 preferred_element_type=jnp.float32)
    hb2 = h2.astype(bf16)
    cs1 = jnp.sum(hn1.astype(jnp.float32), axis=0, keepdims=True)
    mu2 = jnp.dot(cs1 / fN, w2_ref[...], preferred_element_type=jnp.float32)
    var2 = _colsum_sq(hb2, ones8) / fN - mu2 * mu2
    a2f = g2_ref[...] * jax.lax.rsqrt(var2 + EPS)
    a2 = a2f.astype(bf16)
    c2 = (b2_ref[...] - mu2 * a2f).astype(bf16)
    hn2 = jnp.maximum(hb2 * a2 + c2, bf16(0.0))

    b2d = batch_ref[...].reshape(1, N)
    ohT = (b2d == jax.lax.broadcasted_iota(jnp.int32, (G, N), 0)).astype(bf16)
    dn_mm = (((1,), (0,)), ((), ()))
    sums = jax.lax.dot_general(ohT, hn2, dn_mm,
                               preferred_element_type=jnp.float32)
    cnt = jnp.sum(ohT.astype(jnp.float32), axis=1, keepdims=True)
    pooled = sums / jnp.maximum(cnt, 1.0)

    h3 = jnp.maximum(
        jnp.dot(pooled, wfc_ref[...], preferred_element_type=jnp.float32)
        + bfc_ref[...], 0.0)
    dn_t = (((1,), (1,)), ((), ()))
    logits = jax.lax.dot_general(h3, wfc1t_ref[...], dn_t,
                                 preferred_element_type=jnp.float32)
    logits = logits + bfc1_ref[...]
    m = jnp.max(logits, axis=-1, keepdims=True)
    sh = logits - m
    lse = jnp.log(jnp.sum(jnp.exp(sh), axis=-1, keepdims=True))
    out_ref[...] = (sh - lse).T


@functools.partial(jax.jit, static_argnames=("interpret",))
def _run(x, batch, Wg1, g1, b1, Wg2, g2, b2, Wfc, bfc, Wfc1, bfc1,
         interpret=False):
    out_t = pl.pallas_call(
        _fused_kernel,
        out_shape=jax.ShapeDtypeStruct((NUM_CLASSES, G), jnp.float32),
        interpret=interpret,
    )(x, batch, Wg1, g1.reshape(1, HIDDEN), b1.reshape(1, HIDDEN),
      Wg2, g2.reshape(1, HIDDEN), b2.reshape(1, HIDDEN),
      Wfc, bfc.reshape(1, HIDDEN), Wfc1.T, bfc1.reshape(1, NUM_CLASSES))
    return out_t.T


def kernel(x, edge_index, batch, Wg1, bg1, g1, b1, Wg2, bg2, g2, b2,
           Wfc, bfc, Wfc1, bfc1):
    del edge_index, bg1, bg2
    return _run(x, batch, Wg1, g1, b1, Wg2, g2, b2, Wfc, bfc, Wfc1, bfc1)
